# Initial kernel scaffold; baseline (speedup 1.0000x reference)
import jax
import jax.numpy as jnp
from jax.experimental import pallas as pl


def kernel(edge_attr, W0, W1, W2):
    raise NotImplementedError("write your pallas kernel here")


# speedup vs baseline: 1.2596x; 1.2596x over previous
import functools

import jax
import jax.numpy as jnp
from jax import lax
from jax.experimental import pallas as pl
from jax.experimental.pallas import tpu as pltpu
from jax.experimental.pallas import tpu_sc as plsc

EMB = 256
NUM_E = 160000
TBL0, TBL1, TBL2 = 5, 6, 2
NCOMBO = TBL0 * TBL1 * TBL2
WC_ROWS = 64

NC, NS = 2, 16
NW = NC * NS
CHUNK = 160
SUBG = 80
NCHUNKS = NUM_E // CHUNK
BASE_PER_W = NCHUNKS // NW
REM = NCHUNKS - BASE_PER_W * NW


def _table_body(w0_ref, w1_ref, w2_ref, wc_ref):
    r = lax.broadcasted_iota(jnp.int32, (WC_ROWS, 1), 0)
    c0 = r // (TBL1 * TBL2)
    c1 = (r // TBL2) % TBL1
    c2 = r % TBL2
    acc = jnp.zeros((WC_ROWS, EMB), jnp.float32)
    for k in range(TBL0):
        acc = acc + jnp.where(c0 == k, 1.0, 0.0) * w0_ref[k, :][None, :]
    for k in range(TBL1):
        acc = acc + jnp.where(c1 == k, 1.0, 0.0) * w1_ref[k, :][None, :]
    for k in range(TBL2):
        acc = acc + jnp.where((c2 == k) & (c0 < TBL0), 1.0, 0.0) * w2_ref[k, :][None, :]
    wc_ref[...] = acc


def _build_table(w0, w1, w2):
    return pl.pallas_call(
        _table_body,
        out_shape=jax.ShapeDtypeStruct((WC_ROWS, EMB), jnp.float32),
    )(w0, w1, w2)


def _sc_body(a0_hbm, a1_hbm, a2_hbm, wc_hbm, out_hbm,
             a0_v, a1_v, a2_v, idx_v, rows_v, sem):
    wid = lax.axis_index("s") * NC + lax.axis_index("c")
    n_w = BASE_PER_W + jnp.where(wid < REM, 1, 0)

    def chunk_step(t, carry):
        cid = wid + t * NW
        base = cid * CHUNK
        pltpu.sync_copy(a0_hbm.at[pl.ds(base, CHUNK)], a0_v)
        pltpu.sync_copy(a1_hbm.at[pl.ds(base, CHUNK)], a1_v)
        pltpu.sync_copy(a2_hbm.at[pl.ds(base, CHUNK)], a2_v)
        for j in range(CHUNK // 16):
            s = pl.ds(j * 16, 16)
            idx_v[s] = a0_v[s] * (TBL1 * TBL2) + a1_v[s] * TBL2 + a2_v[s]
        copies = []
        for g in range(CHUNK // SUBG):
            gs = pl.ds(g * SUBG, SUBG)
            copies.append(pltpu.async_copy(
                wc_hbm.at[idx_v.at[gs]], rows_v.at[gs], sem))
        for cp in copies:
            cp.wait()
        pltpu.sync_copy(rows_v, out_hbm.at[pl.ds(base, CHUNK)])
        return carry

    lax.fori_loop(0, n_w, chunk_step, 0)


@functools.partial(
    pl.kernel,
    mesh=plsc.VectorSubcoreMesh(core_axis_name="c", subcore_axis_name="s"),
    out_type=jax.ShapeDtypeStruct((NUM_E, EMB), jnp.float32),
    scratch_types=[
        pltpu.VMEM((CHUNK,), jnp.int32),
        pltpu.VMEM((CHUNK,), jnp.int32),
        pltpu.VMEM((CHUNK,), jnp.int32),
        pltpu.VMEM((CHUNK,), jnp.int32),
        pltpu.VMEM((CHUNK, EMB), jnp.float32),
        pltpu.SemaphoreType.DMA,
    ],
)
def _sc_gather(a0_hbm, a1_hbm, a2_hbm, wc_hbm, out_hbm,
               a0_v, a1_v, a2_v, idx_v, rows_v, sem):
    _sc_body(a0_hbm, a1_hbm, a2_hbm, wc_hbm, out_hbm,
             a0_v, a1_v, a2_v, idx_v, rows_v, sem)


def kernel(edge_attr, W0, W1, W2):
    ea = edge_attr.astype(jnp.int32)
    a0 = ea[:, 0]
    a1 = ea[:, 1]
    a2 = ea[:, 2]
    wc = _build_table(W0, W1, W2)
    return _sc_gather(a0, a1, a2, wc)
